# speedup vs baseline: 4.0710x; 4.0710x over previous
import functools

import jax
import jax.numpy as jnp
from jax import lax
from jax.experimental import pallas as pl
from jax.experimental.pallas import tpu as pltpu
from jax.experimental.pallas import tpu_sc as plsc

NC = 2
NS = 16
CH = 128


def _matmul_body(x_ref, w_ref, y_ref):
    y_ref[0] = jnp.dot(x_ref[...], w_ref[0], preferred_element_type=jnp.float32)


def _combine_body(p_ref, b_ref, o_ref):
    o_ref[...] = p_ref[0] + p_ref[1] + b_ref[0][None, :]


def _make_scatter_kernel(n_ch, acc_rows, d_out):
    mesh = plsc.VectorSubcoreMesh(core_axis_name="c", subcore_axis_name="s")
    chunks_per_tile = acc_rows // CH // NS
    slab = acc_rows // NS

    @functools.partial(
        pl.kernel,
        out_type=jax.ShapeDtypeStruct((NC, acc_rows, d_out), jnp.float32),
        mesh=mesh,
        scratch_types=[
            pltpu.VMEM((n_ch, CH), jnp.int32),
            pltpu.VMEM((n_ch, CH), jnp.int32),
            pltpu.VMEM((CH, d_out), jnp.float32),
            pltpu.VMEM_SHARED((acc_rows, d_out), jnp.float32),
            pltpu.SemaphoreType.DMA,
        ],
    )
    def scatter_kernel(yflat, inidx_hbm, outidx_hbm, part_out,
                       inidx_v, outidx_v, rows_v, acc, sem):
        cid = lax.axis_index("c")
        sid = lax.axis_index("s")
        w = cid * NS + sid

        pltpu.sync_copy(inidx_hbm.at[w], inidx_v)
        pltpu.sync_copy(outidx_hbm.at[w], outidx_v)

        def zbody(t, carry):
            rows_v[t // 8, pl.ds((t % 8) * 16, 16)] = jnp.zeros((16,), jnp.float32)
            return carry
        lax.fori_loop(0, CH * (d_out // 16), zbody, 0)
        for i in range(chunks_per_tile):
            pltpu.sync_copy(rows_v, acc.at[pl.ds((sid * chunks_per_tile + i) * CH, CH)])
        plsc.subcore_barrier()

        def chunk(j, carry):
            pltpu.async_copy(yflat.at[inidx_v.at[j]], rows_v, sem).wait()
            pltpu.sync_copy(rows_v, acc.at[outidx_v.at[j]], add=True)
            return carry
        lax.fori_loop(0, n_ch, chunk, 0)
        plsc.subcore_barrier()

        pltpu.sync_copy(acc.at[pl.ds(sid * slab, slab)],
                        part_out.at[cid, pl.ds(sid * slab, slab)])

    return scatter_kernel


def kernel(x_data, rules, rules_count, weights, bias):
    n = x_data.shape[0]
    d_in = x_data.shape[1]
    k3 = weights.shape[0]
    d_out = weights.shape[2]
    r = rules.shape[0]

    blk = 2000
    nb = n // blk
    y = pl.pallas_call(
        _matmul_body,
        grid=(k3, nb),
        in_specs=[
            pl.BlockSpec((blk, d_in), lambda k, j: (j, 0)),
            pl.BlockSpec((1, d_in, d_out), lambda k, j: (k, 0, 0)),
        ],
        out_specs=pl.BlockSpec((1, blk, d_out), lambda k, j: (k, j, 0)),
        out_shape=jax.ShapeDtypeStruct((k3, n, d_out), jnp.float32),
    )(x_data, weights)
    y_flat = y.reshape(k3 * n, d_out)

    nw = NC * NS
    flat_in = rules[:, 0] * n + rules[:, 1]
    out_inds = rules[:, 2]
    per_w = -(-r // nw)
    n_ch = -(-per_w // CH)
    rpad = nw * n_ch * CH
    pad = rpad - r
    flat_in = jnp.concatenate([flat_in, jnp.zeros((pad,), jnp.int32)])
    out_inds = jnp.concatenate([out_inds, jnp.full((pad,), n, jnp.int32)])
    in_idx = flat_in.reshape(nw, n_ch, CH)
    out_idx = out_inds.reshape(nw, n_ch, CH)

    acc_rows = -(-(n + 1) // (CH * NS)) * (CH * NS)

    scatter = _make_scatter_kernel(n_ch, acc_rows, d_out)
    partials = scatter(y_flat, in_idx, out_idx)

    out = pl.pallas_call(
        _combine_body,
        grid=(nb,),
        in_specs=[
            pl.BlockSpec((NC, blk, d_out), lambda j: (0, j, 0)),
            pl.BlockSpec((1, d_out), lambda j: (0, 0)),
        ],
        out_specs=pl.BlockSpec((blk, d_out), lambda j: (j, 0)),
        out_shape=jax.ShapeDtypeStruct((n, d_out), jnp.float32),
    )(partials, bias.reshape(1, d_out))
    return out


# speedup vs baseline: 7.5000x; 1.8423x over previous
import functools

import jax
import jax.numpy as jnp
from jax import lax
from jax.experimental import pallas as pl
from jax.experimental.pallas import tpu as pltpu
from jax.experimental.pallas import tpu_sc as plsc

NC = 2
NS = 16
CH = 128


def _matmul_body(x_ref, w_ref, y_ref):
    y_ref[0] = jnp.dot(x_ref[...], w_ref[0], preferred_element_type=jnp.float32)


def _combine_body(p_ref, b_ref, o_ref):
    o_ref[...] = p_ref[0] + p_ref[1] + b_ref[0][None, :]


def _make_scatter_kernel(n_ch, acc_rows, d_out):
    mesh = plsc.VectorSubcoreMesh(core_axis_name="c", subcore_axis_name="s")
    chunks_per_tile = acc_rows // CH // NS
    slab = acc_rows // NS

    @functools.partial(
        pl.kernel,
        out_type=jax.ShapeDtypeStruct((NC, acc_rows, d_out), jnp.float32),
        mesh=mesh,
        scratch_types=[
            pltpu.VMEM((n_ch // 2, CH), jnp.int32),
            pltpu.VMEM((n_ch // 2, CH), jnp.int32),
            pltpu.VMEM((CH, d_out), jnp.float32),
            pltpu.VMEM((CH, d_out), jnp.float32),
            pltpu.VMEM_SHARED((acc_rows, d_out), jnp.float32),
            pltpu.SemaphoreType.DMA,
            pltpu.SemaphoreType.DMA,
        ],
    )
    def scatter_kernel(yflat, inidx_hbm, outidx_hbm, part_out,
                       inidx_v, outidx_v, rows_v, rows_b, acc, sem, sem_b):
        cid = lax.axis_index("c")
        sid = lax.axis_index("s")
        w = cid * NS + sid
        half = n_ch // 2

        def zbody(t, carry):
            rows_v[t // 8, pl.ds((t % 8) * 16, 16)] = jnp.zeros((16,), jnp.float32)
            return carry
        lax.fori_loop(0, CH * (d_out // 16), zbody, 0)
        for i in range(chunks_per_tile):
            pltpu.sync_copy(rows_v, acc.at[pl.ds((sid * chunks_per_tile + i) * CH, CH)])
        plsc.subcore_barrier()

        for h in range(2):
            pltpu.sync_copy(inidx_hbm.at[w, pl.ds(h * half, half)], inidx_v)
            pltpu.sync_copy(outidx_hbm.at[w, pl.ds(h * half, half)], outidx_v)
            pltpu.async_copy(yflat.at[inidx_v.at[0]], rows_v, sem)

            def pair(p, carry):
                j0 = 2 * p
                j1 = 2 * p + 1
                pltpu.make_async_copy(yflat.at[inidx_v.at[j0]], rows_v, sem).wait()
                pltpu.async_copy(yflat.at[inidx_v.at[j1]], rows_b, sem_b)
                pltpu.sync_copy(rows_v, acc.at[outidx_v.at[j0]], add=True)
                pltpu.make_async_copy(yflat.at[inidx_v.at[j1]], rows_b, sem_b).wait()

                @pl.when(j1 + 1 < half)
                def _():
                    pltpu.async_copy(yflat.at[inidx_v.at[j1 + 1]], rows_v, sem)

                pltpu.sync_copy(rows_b, acc.at[outidx_v.at[j1]], add=True)
                return carry
            lax.fori_loop(0, half // 2, pair, 0)
        plsc.subcore_barrier()

        pltpu.sync_copy(acc.at[pl.ds(sid * slab, slab)],
                        part_out.at[cid, pl.ds(sid * slab, slab)])

    return scatter_kernel


def kernel(x_data, rules, rules_count, weights, bias):
    n = x_data.shape[0]
    d_in = x_data.shape[1]
    k3 = weights.shape[0]
    d_out = weights.shape[2]
    r = rules.shape[0]

    blk = 2000
    nb = n // blk
    y = pl.pallas_call(
        _matmul_body,
        grid=(nb, k3),
        in_specs=[
            pl.BlockSpec((blk, d_in), lambda j, k: (j, 0)),
            pl.BlockSpec((1, d_in, d_out), lambda j, k: (k, 0, 0)),
        ],
        out_specs=pl.BlockSpec((1, blk, d_out), lambda j, k: (k, j, 0)),
        out_shape=jax.ShapeDtypeStruct((k3, n, d_out), jnp.float32),
    )(x_data, weights)
    y_flat = y.reshape(k3 * n, d_out)

    nw = NC * NS
    flat_in = rules[:, 0] * n + rules[:, 1]
    out_inds = rules[:, 2]
    per_w = -(-r // nw)
    n_ch = -(-per_w // CH)
    rpad = nw * n_ch * CH
    pad = rpad - r
    acc_rows = -(-(n + 1) // (CH * NS)) * (CH * NS)
    n_dump = acc_rows - n
    flat_in = jnp.concatenate([flat_in, jnp.arange(pad, dtype=jnp.int32) % n])
    out_inds = jnp.concatenate(
        [out_inds, n + (jnp.arange(pad, dtype=jnp.int32) % n_dump)])
    in_idx = flat_in.reshape(nw, n_ch, CH)
    out_idx = out_inds.reshape(nw, n_ch, CH)

    scatter = _make_scatter_kernel(n_ch, acc_rows, d_out)
    partials = scatter(y_flat, in_idx, out_idx)

    out = pl.pallas_call(
        _combine_body,
        grid=(nb,),
        in_specs=[
            pl.BlockSpec((NC, blk, d_out), lambda j: (0, j, 0)),
            pl.BlockSpec((1, d_out), lambda j: (0, 0)),
        ],
        out_specs=pl.BlockSpec((blk, d_out), lambda j: (j, 0)),
        out_shape=jax.ShapeDtypeStruct((n, d_out), jnp.float32),
    )(partials, bias.reshape(1, d_out))
    return out


# speedup vs baseline: 7.7064x; 1.0275x over previous
import functools

import jax
import jax.numpy as jnp
from jax import lax
from jax.experimental import pallas as pl
from jax.experimental.pallas import tpu as pltpu
from jax.experimental.pallas import tpu_sc as plsc

NC = 2
NS = 16
CH = 128


def _matmul_body(x_ref, w_ref, y_ref):
    y_ref[0] = jnp.dot(x_ref[...], w_ref[0], preferred_element_type=jnp.float32)


def _combine_body(p0_ref, p1_ref, p2_ref, b_ref, o_ref):
    o_ref[...] = (p0_ref[0] + p0_ref[1] + p1_ref[0] + p1_ref[1]
                  + p2_ref[0] + p2_ref[1] + b_ref[0][None, :])


def _make_scatter_kernel(n_ch, acc_rows, d_out):
    mesh = plsc.VectorSubcoreMesh(core_axis_name="c", subcore_axis_name="s")
    chunks_per_tile = acc_rows // CH // NS
    slab = acc_rows // NS

    @functools.partial(
        pl.kernel,
        out_type=jax.ShapeDtypeStruct((NC, acc_rows, d_out), jnp.float32),
        mesh=mesh,
        scratch_types=[
            pltpu.VMEM((n_ch // 2, CH), jnp.int32),
            pltpu.VMEM((n_ch // 2, CH), jnp.int32),
            pltpu.VMEM((CH, d_out), jnp.float32),
            pltpu.VMEM((CH, d_out), jnp.float32),
            pltpu.VMEM_SHARED((acc_rows, d_out), jnp.float32),
            pltpu.SemaphoreType.DMA,
            pltpu.SemaphoreType.DMA,
        ],
    )
    def scatter_kernel(yflat, inidx_hbm, outidx_hbm, part_out,
                       inidx_v, outidx_v, rows_v, rows_b, acc, sem, sem_b):
        cid = lax.axis_index("c")
        sid = lax.axis_index("s")
        w = cid * NS + sid
        half = n_ch // 2

        def zbody(t, carry):
            rows_v[t // 8, pl.ds((t % 8) * 16, 16)] = jnp.zeros((16,), jnp.float32)
            return carry
        lax.fori_loop(0, CH * (d_out // 16), zbody, 0)
        for i in range(chunks_per_tile):
            pltpu.sync_copy(rows_v, acc.at[pl.ds((sid * chunks_per_tile + i) * CH, CH)])
        plsc.subcore_barrier()

        for h in range(2):
            pltpu.sync_copy(inidx_hbm.at[w, h], inidx_v)
            pltpu.sync_copy(outidx_hbm.at[w, h], outidx_v)
            pltpu.async_copy(yflat.at[inidx_v.at[0]], rows_v, sem)

            def pair(p, carry):
                j0 = 2 * p
                j1 = 2 * p + 1
                pltpu.make_async_copy(yflat.at[inidx_v.at[j0]], rows_v, sem).wait()
                pltpu.async_copy(yflat.at[inidx_v.at[j1]], rows_b, sem_b)
                pltpu.sync_copy(rows_v, acc.at[outidx_v.at[j0]], add=True)
                pltpu.make_async_copy(yflat.at[inidx_v.at[j1]], rows_b, sem_b).wait()

                @pl.when(j1 + 1 < half)
                def _():
                    pltpu.async_copy(yflat.at[inidx_v.at[j1 + 1]], rows_v, sem)

                pltpu.sync_copy(rows_b, acc.at[outidx_v.at[j1]], add=True)
                return carry
            lax.fori_loop(0, half // 2, pair, 0)
        plsc.subcore_barrier()

        pltpu.sync_copy(acc.at[pl.ds(sid * slab, slab)],
                        part_out.at[cid, pl.ds(sid * slab, slab)])

    return scatter_kernel


def kernel(x_data, rules, rules_count, weights, bias):
    n = x_data.shape[0]
    d_in = x_data.shape[1]
    k3 = weights.shape[0]
    d_out = weights.shape[2]
    r = rules.shape[0]

    nsplit = 3 if k3 % 3 == 0 and r % k3 == 0 else 1
    ks = k3 // nsplit
    seg = r // k3
    rs = ks * seg

    nw = NC * NS
    per_w = -(-rs // nw)
    n_ch = -(-per_w // (4 * CH)) * 4
    rpad = nw * n_ch * CH
    pad = rpad - rs
    acc_rows = -(-(n + 1) // (CH * NS)) * (CH * NS)
    n_dump = acc_rows - n
    pad_in = jnp.arange(pad, dtype=jnp.int32) % n
    pad_out = n + (jnp.arange(pad, dtype=jnp.int32) % n_dump)

    blk = 2000
    nb = n // blk
    scatter = _make_scatter_kernel(n_ch, acc_rows, d_out)

    partials = []
    for s in range(nsplit):
        y = pl.pallas_call(
            _matmul_body,
            grid=(nb, ks),
            in_specs=[
                pl.BlockSpec((blk, d_in), lambda j, k: (j, 0)),
                pl.BlockSpec((1, d_in, d_out), lambda j, k: (k, 0, 0)),
            ],
            out_specs=pl.BlockSpec((1, blk, d_out), lambda j, k: (k, j, 0)),
            out_shape=jax.ShapeDtypeStruct((ks, n, d_out), jnp.float32),
        )(x_data, lax.slice_in_dim(weights, s * ks, (s + 1) * ks, axis=0))
        y_flat = y.reshape(ks * n, d_out)

        rsl = lax.slice_in_dim(rules, s * rs, (s + 1) * rs, axis=0)
        flat_in = (rsl[:, 0] - s * ks) * n + rsl[:, 1]
        in_idx = jnp.concatenate([flat_in, pad_in]).reshape(nw, 2, n_ch // 2, CH)
        out_idx = jnp.concatenate([rsl[:, 2], pad_out]).reshape(nw, 2, n_ch // 2, CH)

        partials.append(scatter(y_flat, in_idx, out_idx))

    while len(partials) < 3:
        partials.append(jnp.zeros_like(partials[0]))

    pspec = pl.BlockSpec((NC, blk, d_out), lambda j: (0, j, 0))
    out = pl.pallas_call(
        _combine_body,
        grid=(nb,),
        in_specs=[pspec, pspec, pspec,
                  pl.BlockSpec((1, d_out), lambda j: (0, 0))],
        out_specs=pl.BlockSpec((blk, d_out), lambda j: (j, 0)),
        out_shape=jax.ShapeDtypeStruct((n, d_out), jnp.float32),
    )(partials[0], partials[1], partials[2], bias.reshape(1, d_out))
    return out


# speedup vs baseline: 8.3300x; 1.0809x over previous
import functools

import jax
import jax.numpy as jnp
from jax import lax
from jax.experimental import pallas as pl
from jax.experimental.pallas import tpu as pltpu
from jax.experimental.pallas import tpu_sc as plsc

NC = 2
NS = 16
CH = 128


def _matmul_body(x_ref, w_ref, y_ref):
    y_ref[0] = jnp.dot(x_ref[...], w_ref[0], preferred_element_type=jnp.float32)


def _combine_body(p0_ref, p1_ref, p2_ref, b_ref, o_ref):
    o_ref[...] = (p0_ref[0] + p0_ref[1] + p1_ref[0] + p1_ref[1]
                  + p2_ref[0] + p2_ref[1] + b_ref[0][None, :])


def _make_scatter_kernel(n_ch, acc_rows, d_out):
    mesh = plsc.VectorSubcoreMesh(core_axis_name="c", subcore_axis_name="s")
    chunks_per_tile = acc_rows // CH // NS
    slab = acc_rows // NS

    @functools.partial(
        pl.kernel,
        out_type=jax.ShapeDtypeStruct((NC, acc_rows, d_out), jnp.float32),
        mesh=mesh,
        scratch_types=[
            pltpu.VMEM((n_ch, CH), jnp.int32),
            pltpu.VMEM((n_ch, CH), jnp.int32),
            pltpu.VMEM((CH, d_out), jnp.float32),
            pltpu.VMEM((CH, d_out), jnp.float32),
            pltpu.VMEM_SHARED((acc_rows, d_out), jnp.float32),
            pltpu.SemaphoreType.DMA,
            pltpu.SemaphoreType.DMA,
            pltpu.SemaphoreType.DMA,
        ],
    )
    def scatter_kernel(yflat, inidx_hbm, outidx_hbm, part_out,
                       inidx_v, outidx_v, rows_v, rows_b, acc, sem, sem_b, sem_i):
        cid = lax.axis_index("c")
        sid = lax.axis_index("s")
        w = cid * NS + sid

        pltpu.async_copy(inidx_hbm.at[w], inidx_v, sem_i)
        pltpu.async_copy(outidx_hbm.at[w], outidx_v, sem_i)

        def zbody(i, carry):
            for jj in range(d_out // 16):
                rows_v[i, pl.ds(jj * 16, 16)] = jnp.zeros((16,), jnp.float32)
            return carry
        lax.fori_loop(0, CH, zbody, 0)
        for i in range(chunks_per_tile):
            pltpu.async_copy(
                rows_v, acc.at[pl.ds((sid * chunks_per_tile + i) * CH, CH)], sem)
        for i in range(chunks_per_tile):
            pltpu.make_async_copy(
                rows_v, acc.at[pl.ds((sid * chunks_per_tile + i) * CH, CH)], sem).wait()
        pltpu.make_async_copy(inidx_hbm.at[w], inidx_v, sem_i).wait()
        pltpu.make_async_copy(outidx_hbm.at[w], outidx_v, sem_i).wait()
        pltpu.async_copy(yflat.at[inidx_v.at[0]], rows_v, sem)
        plsc.subcore_barrier()

        def pair(p, carry):
            j0 = 2 * p
            j1 = 2 * p + 1
            pltpu.make_async_copy(yflat.at[inidx_v.at[j0]], rows_v, sem).wait()
            pltpu.async_copy(yflat.at[inidx_v.at[j1]], rows_b, sem_b)
            pltpu.sync_copy(rows_v, acc.at[outidx_v.at[j0]], add=True)
            pltpu.make_async_copy(yflat.at[inidx_v.at[j1]], rows_b, sem_b).wait()

            @pl.when(j1 + 1 < n_ch)
            def _():
                pltpu.async_copy(yflat.at[inidx_v.at[j1 + 1]], rows_v, sem)

            pltpu.sync_copy(rows_b, acc.at[outidx_v.at[j1]], add=True)
            return carry
        lax.fori_loop(0, n_ch // 2, pair, 0)
        plsc.subcore_barrier()

        pltpu.sync_copy(acc.at[pl.ds(sid * slab, slab)],
                        part_out.at[cid, pl.ds(sid * slab, slab)])

    return scatter_kernel


def kernel(x_data, rules, rules_count, weights, bias):
    n = x_data.shape[0]
    d_in = x_data.shape[1]
    k3 = weights.shape[0]
    d_out = weights.shape[2]
    r = rules.shape[0]

    nsplit = 3 if k3 % 3 == 0 and r % k3 == 0 else 1
    ks = k3 // nsplit
    seg = r // k3
    rs = ks * seg

    nw = NC * NS
    per_w = -(-rs // nw)
    n_ch = -(-per_w // (4 * CH)) * 4
    rpad = nw * n_ch * CH
    pad = rpad - rs
    acc_rows = -(-(n + 1) // (CH * NS)) * (CH * NS)
    n_dump = acc_rows - n
    pad_in = jnp.arange(pad, dtype=jnp.int32) % n
    pad_out = n + (jnp.arange(pad, dtype=jnp.int32) % n_dump)

    blk = 2000
    nb = n // blk
    scatter = _make_scatter_kernel(n_ch, acc_rows, d_out)

    partials = []
    for s in range(nsplit):
        y = pl.pallas_call(
            _matmul_body,
            grid=(nb, ks),
            in_specs=[
                pl.BlockSpec((blk, d_in), lambda j, k: (j, 0)),
                pl.BlockSpec((1, d_in, d_out), lambda j, k: (k, 0, 0)),
            ],
            out_specs=pl.BlockSpec((1, blk, d_out), lambda j, k: (k, j, 0)),
            out_shape=jax.ShapeDtypeStruct((ks, n, d_out), jnp.float32),
        )(x_data, lax.slice_in_dim(weights, s * ks, (s + 1) * ks, axis=0))
        y_flat = y.reshape(ks * n, d_out)

        rsl = lax.slice_in_dim(rules, s * rs, (s + 1) * rs, axis=0)
        flat_in = (rsl[:, 0] - s * ks) * n + rsl[:, 1]
        in_idx = jnp.concatenate([flat_in, pad_in]).reshape(nw, n_ch, CH)
        out_idx = jnp.concatenate([rsl[:, 2], pad_out]).reshape(nw, n_ch, CH)

        partials.append(scatter(y_flat, in_idx, out_idx))

    while len(partials) < 3:
        partials.append(jnp.zeros_like(partials[0]))

    pspec = pl.BlockSpec((NC, blk, d_out), lambda j: (0, j, 0))
    out = pl.pallas_call(
        _combine_body,
        grid=(nb,),
        in_specs=[pspec, pspec, pspec,
                  pl.BlockSpec((1, d_out), lambda j: (0, 0))],
        out_specs=pl.BlockSpec((blk, d_out), lambda j: (j, 0)),
        out_shape=jax.ShapeDtypeStruct((n, d_out), jnp.float32),
    )(partials[0], partials[1], partials[2], bias.reshape(1, d_out))
    return out
